# baseline (device time: 466527 ns/iter reference)
import jax
import jax.numpy as jnp
from jax import lax
from jax.experimental import pallas as pl
from jax.experimental.pallas import tpu as pltpu

N = 16
B = 64
D = 1024
H = 2048


def kernel(x, Win0, Wout0, Win1, Wout1, Win2, Wout2):
    xb = x.astype(jnp.bfloat16)
    win0 = Win0.astype(jnp.bfloat16)
    win1 = Win1.astype(jnp.bfloat16)
    win2 = Win2.astype(jnp.bfloat16)
    wout0 = Wout0.astype(jnp.bfloat16)
    wout1 = Wout1.astype(jnp.bfloat16)
    wout2 = Wout2.astype(jnp.bfloat16)

    def body(x_ref, w_in0, w_out0, w_in1, w_out1, w_in2, w_out2, out_ref,
             X, P, agb, arb, ag_s, ag_r, rs_s, rs_r):
        my = lax.axis_index("i")
        left = lax.rem(my + N - 1, N)
        right = lax.rem(my + 1, N)

        bar = pltpu.get_barrier_semaphore()
        pl.semaphore_signal(bar, inc=1, device_id=(left,),
                            device_id_type=pl.DeviceIdType.MESH)
        pl.semaphore_signal(bar, inc=1, device_id=(right,),
                            device_id_type=pl.DeviceIdType.MESH)
        pl.semaphore_wait(bar, 2)

        X[pl.ds(my * B, B), :] = x_ref[...]
        agb[0] = x_ref[...]

        def ag_step(h, carry):
            s = lax.rem(h, 2)
            r = lax.rem(h + 1, 2)
            rdma = pltpu.make_async_remote_copy(
                src_ref=agb.at[s],
                dst_ref=agb.at[r],
                send_sem=ag_s.at[s],
                recv_sem=ag_r.at[r],
                device_id=(right,),
                device_id_type=pl.DeviceIdType.MESH,
            )
            rdma.start()
            rdma.wait()
            origin = lax.rem(my + 2 * N - h - 1, N)
            X[pl.ds(origin * B, B), :] = agb[r]
            return carry

        lax.fori_loop(0, N - 1, ag_step, 0)

        def run_layer(w_in, w_out):
            h = jnp.dot(X[...], w_in[...], preferred_element_type=jnp.float32)
            h = jnp.maximum(h, 0.0).astype(jnp.bfloat16)
            P[...] = jnp.dot(h, w_out[...], preferred_element_type=jnp.float32)

            def rs_step(s_, carry):
                snd = lax.rem(s_, 2)
                rcv = lax.rem(s_ + 1, 2)
                c_send = lax.rem(my + 2 * N - s_, N)
                c_recv = lax.rem(my + 2 * N - s_ - 1, N)
                rdma = pltpu.make_async_remote_copy(
                    src_ref=P.at[pl.ds(c_send * B, B), :],
                    dst_ref=arb.at[rcv],
                    send_sem=rs_s.at[snd],
                    recv_sem=rs_r.at[rcv],
                    device_id=(right,),
                    device_id_type=pl.DeviceIdType.MESH,
                )
                rdma.start()
                rdma.wait()
                P[pl.ds(c_recv * B, B), :] = (
                    P[pl.ds(c_recv * B, B), :] + arb[rcv]
                )
                return carry

            lax.fori_loop(0, N - 1, rs_step, 0)

            owned = lax.rem(my + 1, N)
            X[pl.ds(owned * B, B), :] = (
                P[pl.ds(owned * B, B), :].astype(jnp.bfloat16)
            )
            agb[0] = P[pl.ds(owned * B, B), :].astype(jnp.bfloat16)

            def ag2_step(t, carry):
                s = lax.rem(t, 2)
                r = lax.rem(t + 1, 2)
                rdma = pltpu.make_async_remote_copy(
                    src_ref=agb.at[s],
                    dst_ref=agb.at[r],
                    send_sem=ag_s.at[s],
                    recv_sem=ag_r.at[r],
                    device_id=(right,),
                    device_id_type=pl.DeviceIdType.MESH,
                )
                rdma.start()
                rdma.wait()
                g_recv = lax.rem(my + 2 * N - t, N)
                X[pl.ds(g_recv * B, B), :] = agb[r]
                return carry

            lax.fori_loop(0, N - 1, ag2_step, 0)

        run_layer(w_in0, w_out0)
        run_layer(w_in1, w_out1)
        run_layer(w_in2, w_out2)

        out_ref[...] = X[...].astype(jnp.float32)

    return pl.pallas_call(
        body,
        out_shape=jax.ShapeDtypeStruct((N * B, D), jnp.float32),
        in_specs=[pl.BlockSpec(memory_space=pltpu.VMEM)] * 7,
        out_specs=pl.BlockSpec(memory_space=pltpu.VMEM),
        scratch_shapes=[
            pltpu.VMEM((N * B, D), jnp.bfloat16),
            pltpu.VMEM((N * B, D), jnp.float32),
            pltpu.VMEM((2, B, D), jnp.bfloat16),
            pltpu.VMEM((2, B, D), jnp.float32),
            pltpu.SemaphoreType.DMA((2,)),
            pltpu.SemaphoreType.DMA((2,)),
            pltpu.SemaphoreType.DMA((2,)),
            pltpu.SemaphoreType.DMA((2,)),
        ],
        compiler_params=pltpu.CompilerParams(collective_id=0),
    )(xb, win0, wout0, win1, wout1, win2, wout2)


# device time: 235150 ns/iter; 1.9840x vs baseline; 1.9840x over previous
import jax
import jax.numpy as jnp
from jax import lax
from jax.experimental import pallas as pl
from jax.experimental.pallas import tpu as pltpu

N = 16
B = 64
D = 1024
H = 2048


def kernel(x, Win0, Wout0, Win1, Wout1, Win2, Wout2):
    xb = x.astype(jnp.bfloat16)
    win0 = Win0.astype(jnp.bfloat16)
    win1 = Win1.astype(jnp.bfloat16)
    win2 = Win2.astype(jnp.bfloat16)
    wout0 = Wout0.astype(jnp.bfloat16)
    wout1 = Wout1.astype(jnp.bfloat16)
    wout2 = Wout2.astype(jnp.bfloat16)

    def body(x_ref, w_in0, w_out0, w_in1, w_out1, w_in2, w_out2, out_ref,
             X, P, Pb, rbuf, ag_s, ag_r, rs_s, rs_r):
        me = lax.axis_index("i")

        bar = pltpu.get_barrier_semaphore()

        def bar_sig(d, c):
            j = lax.rem(me + d, N)
            pl.semaphore_signal(bar, inc=1, device_id=(j,),
                                device_id_type=pl.DeviceIdType.MESH)
            return c

        lax.fori_loop(1, N, bar_sig, 0)
        pl.semaphore_wait(bar, N - 1)

        def ag_broadcast(d, c):
            j = lax.rem(me + d, N)
            pltpu.make_async_remote_copy(
                src_ref=X.at[pl.ds(me * B, B), :],
                dst_ref=X.at[pl.ds(me * B, B), :],
                send_sem=ag_s.at[j],
                recv_sem=ag_r.at[me],
                device_id=(j,),
                device_id_type=pl.DeviceIdType.MESH,
            ).start()
            return c

        def ag_wait(d, c):
            j = lax.rem(me + d, N)
            desc = pltpu.make_async_remote_copy(
                src_ref=X.at[pl.ds(me * B, B), :],
                dst_ref=X.at[pl.ds(j * B, B), :],
                send_sem=ag_s.at[j],
                recv_sem=ag_r.at[j],
                device_id=(j,),
                device_id_type=pl.DeviceIdType.MESH,
            )
            desc.wait_recv()
            return c

        def ag_wait_send(d, c):
            j = lax.rem(me + d, N)
            desc = pltpu.make_async_remote_copy(
                src_ref=X.at[pl.ds(me * B, B), :],
                dst_ref=X.at[pl.ds(me * B, B), :],
                send_sem=ag_s.at[j],
                recv_sem=ag_r.at[me],
                device_id=(j,),
                device_id_type=pl.DeviceIdType.MESH,
            )
            desc.wait_send()
            return c

        def run_allgather():
            lax.fori_loop(1, N, ag_broadcast, 0)
            lax.fori_loop(1, N, ag_wait, 0)
            lax.fori_loop(1, N, ag_wait_send, 0)

        X[pl.ds(me * B, B), :] = x_ref[...]
        run_allgather()

        def run_layer(w_in, w_out):
            h = jnp.dot(X[...], w_in[...], preferred_element_type=jnp.float32)
            h = jnp.maximum(h, 0.0).astype(jnp.bfloat16)
            P[...] = jnp.dot(h, w_out[...], preferred_element_type=jnp.float32)
            Pb[...] = P[...].astype(jnp.bfloat16)

            def rs_send(d, c):
                j = lax.rem(me + d, N)
                pltpu.make_async_remote_copy(
                    src_ref=Pb.at[pl.ds(j * B, B), :],
                    dst_ref=rbuf.at[me],
                    send_sem=rs_s.at[j],
                    recv_sem=rs_r.at[me],
                    device_id=(j,),
                    device_id_type=pl.DeviceIdType.MESH,
                ).start()
                return c

            lax.fori_loop(1, N, rs_send, 0)

            def rs_wait_acc(d, acc):
                j = lax.rem(me + d, N)
                desc = pltpu.make_async_remote_copy(
                    src_ref=Pb.at[pl.ds(me * B, B), :],
                    dst_ref=rbuf.at[j],
                    send_sem=rs_s.at[j],
                    recv_sem=rs_r.at[j],
                    device_id=(j,),
                    device_id_type=pl.DeviceIdType.MESH,
                )
                desc.wait_recv()
                return acc + rbuf[j].astype(jnp.float32)

            acc = lax.fori_loop(
                1, N, rs_wait_acc, P[pl.ds(me * B, B), :]
            )

            def rs_wait_send(d, c):
                j = lax.rem(me + d, N)
                desc = pltpu.make_async_remote_copy(
                    src_ref=Pb.at[pl.ds(j * B, B), :],
                    dst_ref=rbuf.at[me],
                    send_sem=rs_s.at[j],
                    recv_sem=rs_r.at[me],
                    device_id=(j,),
                    device_id_type=pl.DeviceIdType.MESH,
                )
                desc.wait_send()
                return c

            lax.fori_loop(1, N, rs_wait_send, 0)

            X[pl.ds(me * B, B), :] = acc.astype(jnp.bfloat16)
            run_allgather()

        run_layer(w_in0, w_out0)
        run_layer(w_in1, w_out1)
        run_layer(w_in2, w_out2)

        out_ref[...] = X[...].astype(jnp.float32)

    return pl.pallas_call(
        body,
        out_shape=jax.ShapeDtypeStruct((N * B, D), jnp.float32),
        in_specs=[pl.BlockSpec(memory_space=pltpu.VMEM)] * 7,
        out_specs=pl.BlockSpec(memory_space=pltpu.VMEM),
        scratch_shapes=[
            pltpu.VMEM((N * B, D), jnp.bfloat16),
            pltpu.VMEM((N * B, D), jnp.float32),
            pltpu.VMEM((N * B, D), jnp.bfloat16),
            pltpu.VMEM((N, B, D), jnp.bfloat16),
            pltpu.SemaphoreType.DMA((N,)),
            pltpu.SemaphoreType.DMA((N,)),
            pltpu.SemaphoreType.DMA((N,)),
            pltpu.SemaphoreType.DMA((N,)),
        ],
        compiler_params=pltpu.CompilerParams(collective_id=0),
    )(xb, win0, wout0, win1, wout1, win2, wout2)


# device time: 216947 ns/iter; 2.1504x vs baseline; 1.0839x over previous
import jax
import jax.numpy as jnp
from jax import lax
from jax.experimental import pallas as pl
from jax.experimental.pallas import tpu as pltpu

N = 16
B = 64
D = 1024
H = 2048
G = 4
NG = N // G


def kernel(x, Win0, Wout0, Win1, Wout1, Win2, Wout2):
    xb = x.astype(jnp.bfloat16)
    win0 = Win0.astype(jnp.bfloat16)
    win1 = Win1.astype(jnp.bfloat16)
    win2 = Win2.astype(jnp.bfloat16)
    wout0 = Wout0.astype(jnp.bfloat16)
    wout1 = Wout1.astype(jnp.bfloat16)
    wout2 = Wout2.astype(jnp.bfloat16)

    def body(x_ref, w_in0, w_out0, w_in1, w_out1, w_in2, w_out2, out_ref,
             X, P, Pb, rbuf, accb, ag_s, ag_r, rs_s, rs_r):
        me = lax.axis_index("i")

        bar = pltpu.get_barrier_semaphore()

        def bar_sig(d, c):
            j = lax.rem(me + d, N)
            pl.semaphore_signal(bar, inc=1, device_id=(j,),
                                device_id_type=pl.DeviceIdType.MESH)
            return c

        lax.fori_loop(1, N, bar_sig, 0)
        pl.semaphore_wait(bar, N - 1)

        def ag_send_all():
            for j in range(N):
                @pl.when(j != me)
                def _():
                    pltpu.make_async_remote_copy(
                        src_ref=X.at[pl.ds(me * B, B), :],
                        dst_ref=X.at[pl.ds(me * B, B), :],
                        send_sem=ag_s.at[j],
                        recv_sem=ag_r.at[me],
                        device_id=(j,),
                        device_id_type=pl.DeviceIdType.MESH,
                    ).start()

        def ag_wait_chunk(j):
            @pl.when(j != me)
            def _():
                pltpu.make_async_remote_copy(
                    src_ref=X.at[pl.ds(me * B, B), :],
                    dst_ref=X.at[pl.ds(j * B, B), :],
                    send_sem=ag_s.at[j],
                    recv_sem=ag_r.at[j],
                    device_id=(j,),
                    device_id_type=pl.DeviceIdType.MESH,
                ).wait_recv()

        def wait_send_all(sems):
            for j in range(N):
                @pl.when(j != me)
                def _():
                    pltpu.make_async_remote_copy(
                        src_ref=X.at[pl.ds(me * B, B), :],
                        dst_ref=X.at[pl.ds(me * B, B), :],
                        send_sem=sems.at[j],
                        recv_sem=ag_r.at[me],
                        device_id=(j,),
                        device_id_type=pl.DeviceIdType.MESH,
                    ).wait_send()

        X[pl.ds(me * B, B), :] = x_ref[...]
        ag_send_all()

        def run_layer(w_in, w_out):
            for g in range(NG):
                for j in range(g * G, (g + 1) * G):
                    ag_wait_chunk(j)
                if g == 1:
                    wait_send_all(ag_s)
                r0 = g * G * B
                hg = jnp.dot(X[r0:r0 + G * B, :], w_in[...],
                             preferred_element_type=jnp.float32)
                hg = jnp.maximum(hg, 0.0).astype(jnp.bfloat16)
                pg = jnp.dot(hg, w_out[...],
                             preferred_element_type=jnp.float32)
                P[r0:r0 + G * B, :] = pg
                Pb[r0:r0 + G * B, :] = pg.astype(jnp.bfloat16)

                for j in range(g * G, (g + 1) * G):
                    @pl.when(j != me)
                    def _():
                        pltpu.make_async_remote_copy(
                            src_ref=Pb.at[pl.ds(j * B, B), :],
                            dst_ref=rbuf.at[me],
                            send_sem=rs_s.at[j],
                            recv_sem=rs_r.at[me],
                            device_id=(j,),
                            device_id_type=pl.DeviceIdType.MESH,
                        ).start()

            accb[...] = P[pl.ds(me * B, B), :]
            for j in range(N):
                @pl.when(j != me)
                def _():
                    pltpu.make_async_remote_copy(
                        src_ref=Pb.at[pl.ds(me * B, B), :],
                        dst_ref=rbuf.at[j],
                        send_sem=rs_s.at[j],
                        recv_sem=rs_r.at[j],
                        device_id=(j,),
                        device_id_type=pl.DeviceIdType.MESH,
                    ).wait_recv()
                    accb[...] += rbuf[j].astype(jnp.float32)

            wait_send_all(rs_s)
            X[pl.ds(me * B, B), :] = accb[...].astype(jnp.bfloat16)
            ag_send_all()

        run_layer(w_in0, w_out0)
        run_layer(w_in1, w_out1)
        run_layer(w_in2, w_out2)

        for j in range(N):
            ag_wait_chunk(j)
        wait_send_all(ag_s)
        out_ref[...] = X[...].astype(jnp.float32)

    return pl.pallas_call(
        body,
        out_shape=jax.ShapeDtypeStruct((N * B, D), jnp.float32),
        in_specs=[pl.BlockSpec(memory_space=pltpu.VMEM)] * 7,
        out_specs=pl.BlockSpec(memory_space=pltpu.VMEM),
        scratch_shapes=[
            pltpu.VMEM((N * B, D), jnp.bfloat16),
            pltpu.VMEM((N * B, D), jnp.float32),
            pltpu.VMEM((N * B, D), jnp.bfloat16),
            pltpu.VMEM((N, B, D), jnp.bfloat16),
            pltpu.VMEM((B, D), jnp.float32),
            pltpu.SemaphoreType.DMA((N,)),
            pltpu.SemaphoreType.DMA((N,)),
            pltpu.SemaphoreType.DMA((N,)),
            pltpu.SemaphoreType.DMA((N,)),
        ],
        compiler_params=pltpu.CompilerParams(collective_id=0),
    )(xb, win0, wout0, win1, wout1, win2, wout2)


# device time: 216842 ns/iter; 2.1515x vs baseline; 1.0005x over previous
import jax
import jax.numpy as jnp
from jax import lax
from jax.experimental import pallas as pl
from jax.experimental.pallas import tpu as pltpu

N = 16
B = 64
D = 1024
H = 2048
G = 4
NG = N // G


def kernel(x, Win0, Wout0, Win1, Wout1, Win2, Wout2):
    xb = x.astype(jnp.bfloat16)
    win0 = Win0.astype(jnp.bfloat16)
    win1 = Win1.astype(jnp.bfloat16)
    win2 = Win2.astype(jnp.bfloat16)
    wout0 = Wout0.astype(jnp.bfloat16)
    wout1 = Wout1.astype(jnp.bfloat16)
    wout2 = Wout2.astype(jnp.bfloat16)

    def body(x_ref, w_in0, w_out0, w_in1, w_out1, w_in2, w_out2, out_ref,
             X, Pb, rbuf, accb, ag_s, ag_r, rs_s, rs_r):
        me = lax.axis_index("i")

        bar = pltpu.get_barrier_semaphore()

        def bar_sig(d, c):
            j = lax.rem(me + d, N)
            pl.semaphore_signal(bar, inc=1, device_id=(j,),
                                device_id_type=pl.DeviceIdType.MESH)
            return c

        lax.fori_loop(1, N, bar_sig, 0)
        pl.semaphore_wait(bar, N - 1)

        def ag_send_all():
            for j in range(N):
                @pl.when(j != me)
                def _():
                    pltpu.make_async_remote_copy(
                        src_ref=X.at[pl.ds(me * B, B), :],
                        dst_ref=X.at[pl.ds(me * B, B), :],
                        send_sem=ag_s.at[j],
                        recv_sem=ag_r.at[me],
                        device_id=(j,),
                        device_id_type=pl.DeviceIdType.MESH,
                    ).start()

        def ag_wait_chunk(j):
            @pl.when(j != me)
            def _():
                pltpu.make_async_remote_copy(
                    src_ref=X.at[pl.ds(me * B, B), :],
                    dst_ref=X.at[pl.ds(j * B, B), :],
                    send_sem=ag_s.at[j],
                    recv_sem=ag_r.at[j],
                    device_id=(j,),
                    device_id_type=pl.DeviceIdType.MESH,
                ).wait_recv()

        def wait_send_all(sems):
            for j in range(N):
                @pl.when(j != me)
                def _():
                    pltpu.make_async_remote_copy(
                        src_ref=X.at[pl.ds(me * B, B), :],
                        dst_ref=X.at[pl.ds(me * B, B), :],
                        send_sem=sems.at[j],
                        recv_sem=ag_r.at[me],
                        device_id=(j,),
                        device_id_type=pl.DeviceIdType.MESH,
                    ).wait_send()

        X[pl.ds(me * B, B), :] = x_ref[...]
        ag_send_all()

        def run_layer(w_in, w_out):
            for g in range(NG):
                for j in range(g * G, (g + 1) * G):
                    ag_wait_chunk(j)
                if g == 1:
                    wait_send_all(ag_s)
                r0 = g * G * B
                hg = jnp.dot(X[r0:r0 + G * B, :], w_in[...],
                             preferred_element_type=jnp.float32)
                hg = jnp.maximum(hg, 0.0).astype(jnp.bfloat16)
                pg = jnp.dot(hg, w_out[...],
                             preferred_element_type=jnp.float32)
                Pb[r0:r0 + G * B, :] = pg.astype(jnp.bfloat16)

                for j in range(g * G, (g + 1) * G):
                    @pl.when(j != me)
                    def _():
                        pltpu.make_async_remote_copy(
                            src_ref=Pb.at[pl.ds(j * B, B), :],
                            dst_ref=rbuf.at[me],
                            send_sem=rs_s.at[j],
                            recv_sem=rs_r.at[me],
                            device_id=(j,),
                            device_id_type=pl.DeviceIdType.MESH,
                        ).start()

            accb[...] = Pb[pl.ds(me * B, B), :].astype(jnp.float32)
            for j in range(N):
                @pl.when(j != me)
                def _():
                    pltpu.make_async_remote_copy(
                        src_ref=Pb.at[pl.ds(me * B, B), :],
                        dst_ref=rbuf.at[j],
                        send_sem=rs_s.at[j],
                        recv_sem=rs_r.at[j],
                        device_id=(j,),
                        device_id_type=pl.DeviceIdType.MESH,
                    ).wait_recv()
                    accb[...] += rbuf[j].astype(jnp.float32)

            wait_send_all(rs_s)
            X[pl.ds(me * B, B), :] = accb[...].astype(jnp.bfloat16)
            ag_send_all()

        run_layer(w_in0, w_out0)
        run_layer(w_in1, w_out1)
        run_layer(w_in2, w_out2)

        for j in range(N):
            ag_wait_chunk(j)
        wait_send_all(ag_s)
        out_ref[...] = X[...].astype(jnp.float32)

    return pl.pallas_call(
        body,
        out_shape=jax.ShapeDtypeStruct((N * B, D), jnp.float32),
        in_specs=[pl.BlockSpec(memory_space=pltpu.VMEM)] * 7,
        out_specs=pl.BlockSpec(memory_space=pltpu.VMEM),
        scratch_shapes=[
            pltpu.VMEM((N * B, D), jnp.bfloat16),
            pltpu.VMEM((N * B, D), jnp.bfloat16),
            pltpu.VMEM((N, B, D), jnp.bfloat16),
            pltpu.VMEM((B, D), jnp.float32),
            pltpu.SemaphoreType.DMA((N,)),
            pltpu.SemaphoreType.DMA((N,)),
            pltpu.SemaphoreType.DMA((N,)),
            pltpu.SemaphoreType.DMA((N,)),
        ],
        compiler_params=pltpu.CompilerParams(collective_id=0),
    )(xb, win0, wout0, win1, wout1, win2, wout2)


# device time: 197560 ns/iter; 2.3614x vs baseline; 1.0976x over previous
import jax
import jax.numpy as jnp
from jax import lax
from jax.experimental import pallas as pl
from jax.experimental.pallas import tpu as pltpu

N = 16
B = 64
D = 1024
H = 2048
G = 4
NG = N // G


def kernel(x, Win0, Wout0, Win1, Wout1, Win2, Wout2):
    xb = x.astype(jnp.bfloat16)

    def body(x_ref, w_in0, w_out0, w_in1, w_out1, w_in2, w_out2, out_ref,
             X, Pb, rbuf, accb, win_st, wout_st, winb, woutb,
             ag_s, ag_r, rs_s, rs_r, wdma):
        me = lax.axis_index("i")
        w_hbm = ((w_in0, w_out0), (w_in1, w_out1), (w_in2, w_out2))

        def wstage_start(layer):
            pltpu.make_async_copy(w_hbm[layer][0], win_st, wdma.at[0]).start()
            pltpu.make_async_copy(w_hbm[layer][1], wout_st, wdma.at[1]).start()

        def wstage_finish(layer):
            slot = layer % 2
            pltpu.make_async_copy(w_hbm[layer][0], win_st, wdma.at[0]).wait()
            pltpu.make_async_copy(w_hbm[layer][1], wout_st, wdma.at[1]).wait()
            winb[slot] = win_st[...].astype(jnp.bfloat16)
            woutb[slot] = wout_st[...].astype(jnp.bfloat16)

        wstage_start(0)

        bar = pltpu.get_barrier_semaphore()

        def bar_sig(d, c):
            j = lax.rem(me + d, N)
            pl.semaphore_signal(bar, inc=1, device_id=(j,),
                                device_id_type=pl.DeviceIdType.MESH)
            return c

        lax.fori_loop(1, N, bar_sig, 0)
        pl.semaphore_wait(bar, N - 1)

        def ag_send_all():
            for j in range(N):
                @pl.when(j != me)
                def _():
                    pltpu.make_async_remote_copy(
                        src_ref=X.at[pl.ds(me * B, B), :],
                        dst_ref=X.at[pl.ds(me * B, B), :],
                        send_sem=ag_s.at[j],
                        recv_sem=ag_r.at[me],
                        device_id=(j,),
                        device_id_type=pl.DeviceIdType.MESH,
                    ).start()

        def ag_wait_chunk(j):
            @pl.when(j != me)
            def _():
                pltpu.make_async_remote_copy(
                    src_ref=X.at[pl.ds(me * B, B), :],
                    dst_ref=X.at[pl.ds(j * B, B), :],
                    send_sem=ag_s.at[j],
                    recv_sem=ag_r.at[j],
                    device_id=(j,),
                    device_id_type=pl.DeviceIdType.MESH,
                ).wait_recv()

        def wait_send_all(sems):
            for j in range(N):
                @pl.when(j != me)
                def _():
                    pltpu.make_async_remote_copy(
                        src_ref=X.at[pl.ds(me * B, B), :],
                        dst_ref=X.at[pl.ds(me * B, B), :],
                        send_sem=sems.at[j],
                        recv_sem=ag_r.at[me],
                        device_id=(j,),
                        device_id_type=pl.DeviceIdType.MESH,
                    ).wait_send()

        X[pl.ds(me * B, B), :] = x_ref[...]
        ag_send_all()

        wstage_finish(0)
        wstage_start(1)

        def run_layer(layer):
            slot = layer % 2
            for g in range(NG):
                for j in range(g * G, (g + 1) * G):
                    ag_wait_chunk(j)
                if g == 1:
                    wait_send_all(ag_s)
                if g == 2 and layer < 2:
                    wstage_finish(layer + 1)
                    if layer < 1:
                        wstage_start(layer + 2)
                r0 = g * G * B
                hg = jnp.dot(X[r0:r0 + G * B, :], winb[slot],
                             preferred_element_type=jnp.float32)
                hg = jnp.maximum(hg, 0.0).astype(jnp.bfloat16)
                pg = jnp.dot(hg, woutb[slot],
                             preferred_element_type=jnp.float32)
                Pb[r0:r0 + G * B, :] = pg.astype(jnp.bfloat16)

                for j in range(g * G, (g + 1) * G):
                    @pl.when(j != me)
                    def _():
                        pltpu.make_async_remote_copy(
                            src_ref=Pb.at[pl.ds(j * B, B), :],
                            dst_ref=rbuf.at[me],
                            send_sem=rs_s.at[j],
                            recv_sem=rs_r.at[me],
                            device_id=(j,),
                            device_id_type=pl.DeviceIdType.MESH,
                        ).start()

            accb[...] = Pb[pl.ds(me * B, B), :].astype(jnp.float32)
            for j in range(N):
                @pl.when(j != me)
                def _():
                    pltpu.make_async_remote_copy(
                        src_ref=Pb.at[pl.ds(me * B, B), :],
                        dst_ref=rbuf.at[j],
                        send_sem=rs_s.at[j],
                        recv_sem=rs_r.at[j],
                        device_id=(j,),
                        device_id_type=pl.DeviceIdType.MESH,
                    ).wait_recv()
                    accb[...] += rbuf[j].astype(jnp.float32)

            wait_send_all(rs_s)
            X[pl.ds(me * B, B), :] = accb[...].astype(jnp.bfloat16)
            ag_send_all()

        run_layer(0)
        run_layer(1)
        run_layer(2)

        for j in range(N):
            ag_wait_chunk(j)
        wait_send_all(ag_s)
        out_ref[...] = X[...].astype(jnp.float32)

    return pl.pallas_call(
        body,
        out_shape=jax.ShapeDtypeStruct((N * B, D), jnp.float32),
        in_specs=[pl.BlockSpec(memory_space=pltpu.VMEM)]
        + [pl.BlockSpec(memory_space=pl.ANY)] * 6,
        out_specs=pl.BlockSpec(memory_space=pltpu.VMEM),
        scratch_shapes=[
            pltpu.VMEM((N * B, D), jnp.bfloat16),
            pltpu.VMEM((N * B, D), jnp.bfloat16),
            pltpu.VMEM((N, B, D), jnp.bfloat16),
            pltpu.VMEM((B, D), jnp.float32),
            pltpu.VMEM((D, H), jnp.float32),
            pltpu.VMEM((H, D), jnp.float32),
            pltpu.VMEM((2, D, H), jnp.bfloat16),
            pltpu.VMEM((2, H, D), jnp.bfloat16),
            pltpu.SemaphoreType.DMA((N,)),
            pltpu.SemaphoreType.DMA((N,)),
            pltpu.SemaphoreType.DMA((N,)),
            pltpu.SemaphoreType.DMA((N,)),
            pltpu.SemaphoreType.DMA((2,)),
        ],
        compiler_params=pltpu.CompilerParams(
            collective_id=0, vmem_limit_bytes=60 * 1024 * 1024),
    )(xb, Win0, Wout0, Win1, Wout1, Win2, Wout2)
